# baseline (speedup 1.0000x reference)
import functools

import jax
import jax.numpy as jnp
from jax import lax
from jax.experimental import pallas as pl
from jax.experimental.pallas import tpu as pltpu
from jax.experimental.pallas import tpu_sc as plsc

DIM = 32
NC = 2
NS = 16
NW = NC * NS
LANES = 16
M = 16
LREC = 20
UPW = 32
UPG = LREC * M
CH = 80

_i32 = jnp.int32
_f32 = jnp.float32


def _proj_body(x_ref, w_ref, s1_ref, s3_ref):
    y = jax.lax.dot_general(w_ref[...], x_ref[...],
                            (((1,), (1,)), ((), ())),
                            preferred_element_type=_f32)
    s1_ref[...] = y[0, :]
    s3_ref[...] = y[1, :]


def _projections(ent, w1, w3):
    n = ent.shape[0]
    blk = 8192
    grid = (n + blk - 1) // blk
    w13 = jnp.stack([w1, w3])
    return pl.pallas_call(
        _proj_body,
        grid=(grid,),
        in_specs=[
            pl.BlockSpec((blk, DIM), lambda g: (g, 0)),
            pl.BlockSpec((2, DIM), lambda g: (0, 0)),
        ],
        out_specs=[pl.BlockSpec((blk,), lambda g: (g,))] * 2,
        out_shape=[jax.ShapeDtypeStruct((n,), _f32)] * 2,
    )(ent, w13)


def _rel_body(x_ref, w2_ref, s2_ref):
    s2_ref[...] = jnp.sum(x_ref[...] * w2_ref[0, :][None, :], axis=1)


def _rel_projection(relp, w2):
    n = relp.shape[0]
    return pl.pallas_call(
        _rel_body,
        out_shape=jax.ShapeDtypeStruct((n,), _f32),
    )(relp, w2.reshape(1, DIM))


def _sc_body(h0, t0, r0, h1, t1, r1, recf, itemsf, wcat, s1t, s3t, s2t, ent4,
             out_hbm, idxb, gtep, s1b, s3b, s2l, itrowp, recb, pidxb, subb,
             uemb, wbuf, pbuf, sbuf, outb, sem):
    wid = lax.axis_index("s") * NC + lax.axis_index("c")
    iota16 = lax.iota(_i32, LANES)

    pltpu.sync_copy(wcat, wbuf)
    pltpu.sync_copy(s2t, s2l)
    pltpu.sync_copy(recf.at[pl.ds(wid * 640, 640)], recb)
    pltpu.sync_copy(itemsf.at[pl.ds(wid * UPW, UPW)], idxb.at[pl.ds(0, UPW)])

    for g in range(2):
        iv = idxb[pl.ds(g * LANES, LANES)]
        pidxb[pl.ds(g * LANES, LANES)] = lax.shift_right_logical(iv, 2)
        subb[pl.ds(g * LANES, LANES)] = lax.shift_left(
            jnp.bitwise_and(iv, jnp.full((LANES,), 3, _i32)), 5)
    pltpu.async_copy(ent4.at[pidxb.at[pl.ds(0, UPW)]], itrowp, sem).wait()

    def _fitem(u, o):
        subspl = plsc.load_gather(subb, [jnp.full((LANES,), u, _i32)])
        c0 = subspl + iota16
        rowspl = jnp.full((LANES,), u, _i32)
        i0 = plsc.load_gather(itrowp, [rowspl, c0])
        i1 = plsc.load_gather(itrowp, [rowspl, c0 + LANES])
        uemb[pl.ds(1024 + u * DIM, LANES)] = i0
        uemb[pl.ds(1024 + u * DIM + LANES, LANES)] = i1
        return o

    lax.fori_loop(0, UPW, _fitem, 0)

    b_v = wbuf[...]

    def _recsub(g, o):
        iv = recb[pl.ds(g * LANES, LANES)]
        pidxb[pl.ds(g * LANES, LANES)] = lax.shift_right_logical(iv, 2)
        subb[pl.ds(g * LANES, LANES)] = lax.shift_left(
            jnp.bitwise_and(iv, jnp.full((LANES,), 3, _i32)), 5)
        return o

    lax.fori_loop(0, 40, _recsub, 0)
    for half in range(2):
        hs = []
        for c in range(4):
            hs.append(pltpu.async_copy(
                ent4.at[pidxb.at[pl.ds(half * 320 + c * CH, CH)]],
                gtep.at[pl.ds(c * CH, CH)], sem))
        for hnd in hs:
            hnd.wait()

        def _recuser(i, _, half=half):
            def _recl(l, acc):
                row = i * LREC + l
                subspl = plsc.load_gather(
                    subb, [jnp.full((LANES,), half * 320 + row, _i32)])
                c0 = subspl + iota16
                rowspl = jnp.full((LANES,), row, _i32)
                return (acc[0] + plsc.load_gather(gtep, [rowspl, c0]),
                        acc[1] + plsc.load_gather(gtep, [rowspl, c0 + LANES]))

            a0, a1 = lax.fori_loop(
                0, LREC, _recl,
                (jnp.zeros((LANES,), _f32), jnp.zeros((LANES,), _f32)))
            u = half * LANES + i
            uemb[pl.ds(u * DIM, LANES)] = a0
            uemb[pl.ds(u * DIM + LANES, LANES)] = a1
            return _

        lax.fori_loop(0, 16, _recuser, 0)

    heads = (h0, h1)
    tails = (t0, t1)
    rels = (r0, r1)

    def _user(u_loc, _):
        off = (wid * UPW + u_loc) * UPG
        hs = []
        for h in range(2):
            hs.append(pltpu.async_copy(heads[h].at[pl.ds(off, UPG)],
                                       idxb.at[pl.ds(h * 960, UPG)], sem))
            hs.append(pltpu.async_copy(tails[h].at[pl.ds(off, UPG)],
                                       idxb.at[pl.ds(h * 960 + 320, UPG)], sem))
            hs.append(pltpu.async_copy(rels[h].at[pl.ds(off, UPG)],
                                       idxb.at[pl.ds(h * 960 + 640, UPG)], sem))
        for hnd in hs:
            hnd.wait()

        c3v = jnp.full((LANES,), 3, _i32)

        def _pk(g, o):
            h = g // 20
            gg = g - h * 20
            iv = idxb[pl.ds(h * 960 + 320 + gg * LANES, LANES)]
            pidxb[pl.ds(g * LANES, LANES)] = lax.shift_right_logical(iv, 2)
            subb[pl.ds(g * LANES, LANES)] = lax.shift_left(
                jnp.bitwise_and(iv, c3v), 5)
            return o

        lax.fori_loop(0, 40, _pk, 0)

        hs = []
        for h in range(2):
            for c in range(4):
                hs.append(pltpu.async_copy(
                    ent4.at[pidxb.at[pl.ds(h * 320 + c * CH, CH)]],
                    gtep.at[pl.ds(h * 320 + c * CH, CH)], sem))
                hs.append(pltpu.async_copy(
                    s1t.at[idxb.at[pl.ds(h * 960 + c * CH, CH)]],
                    s1b.at[pl.ds(h * 320 + c * CH, CH)], sem))
                hs.append(pltpu.async_copy(
                    s3t.at[idxb.at[pl.ds(h * 960 + 320 + c * CH, CH)]],
                    s3b.at[pl.ds(h * 320 + c * CH, CH)], sem))
        for hnd in hs:
            hnd.wait()

        acc0 = uemb[pl.ds(u_loc * DIM, LANES)]
        acc1 = uemb[pl.ds(u_loc * DIM + LANES, LANES)]

        for h in range(2):

            def _lstep(l, carry, h=h):
                a0, a1 = carry
                rbase = l * M

                s1 = s1b[pl.ds(h * 320 + rbase, M)]
                s3 = s3b[pl.ds(h * 320 + rbase, M)]
                relidx = idxb[pl.ds(h * 960 + 640 + rbase, M)]
                s2 = plsc.load_gather(s2l, [relidx])
                logit = s1 + s2 + s3 + b_v
                sig = 1.0 / (1.0 + jnp.exp(-logit))
                e = jnp.exp(sig)
                p = e / jnp.sum(e)
                pbuf[...] = p
                sbuf[...] = subb[pl.ds(h * 320 + rbase, M)]

                def _mstep(mj, c3):
                    b0, b1 = c3
                    for q in range(2):
                        m = mj * 2 + q
                        mspl = jnp.full((LANES,), m, _i32)
                        pm = plsc.load_gather(pbuf, [mspl])
                        c0 = plsc.load_gather(sbuf, [mspl]) + iota16
                        rowspl = jnp.full((LANES,), h * 320 + rbase + m, _i32)
                        b0 = b0 + pm * plsc.load_gather(gtep, [rowspl, c0])
                        b1 = b1 + pm * plsc.load_gather(gtep, [rowspl, c0 + LANES])
                    return (b0, b1)

                return lax.fori_loop(0, 8, _mstep, (a0, a1))

            acc0, acc1 = lax.fori_loop(0, LREC, _lstep, (acc0, acc1))

        uemb[pl.ds(u_loc * DIM, LANES)] = acc0
        uemb[pl.ds(u_loc * DIM + LANES, LANES)] = acc1
        return _

    lax.fori_loop(0, UPW, _user, 0)

    for g in range(2):

        def _fu(i, o, g=g):
            u = g * LANES + i
            a0 = uemb[pl.ds(u * DIM, LANES)]
            a1 = uemb[pl.ds(u * DIM + LANES, LANES)]
            i0 = uemb[pl.ds(1024 + u * DIM, LANES)]
            i1 = uemb[pl.ds(1024 + u * DIM + LANES, LANES)]
            dot = jnp.sum(a0 * i0 + a1 * i1)
            return jnp.where(iota16 == i, dot, o)

        dots = lax.fori_loop(0, LANES, _fu, jnp.zeros((LANES,), _f32))
        outb[pl.ds(g * LANES, LANES)] = 1.0 / (1.0 + jnp.exp(-dots))

    pltpu.sync_copy(outb, out_hbm.at[pl.ds(wid * UPW, UPW)])


def kernel(pairs, records, ripple_heads, ripple_relations, ripple_tails,
           entity_emb, relation_emb, W_w, W_b):
    Bn, Ln = records.shape
    items = pairs[:, 1].astype(_i32)
    recf = records.reshape(-1).astype(_i32)
    h0 = ripple_heads[0].reshape(-1).astype(_i32)
    h1 = ripple_heads[1].reshape(-1).astype(_i32)
    t0 = ripple_tails[0].reshape(-1).astype(_i32)
    t1 = ripple_tails[1].reshape(-1).astype(_i32)
    r0 = ripple_relations[0].reshape(-1).astype(_i32)
    r1 = ripple_relations[1].reshape(-1).astype(_i32)
    wcat = jnp.full((LANES,), W_b[0], _f32)
    relp = jnp.pad(relation_emb, ((0, 1024 - relation_emb.shape[0]), (0, 0)))

    ent4 = entity_emb.reshape(250000, 128)
    s1t, s3t = _projections(entity_emb, W_w[:DIM, 0], W_w[2 * DIM:, 0])
    s2t = _rel_projection(relp, W_w[DIM:2 * DIM, 0])

    mesh = plsc.VectorSubcoreMesh(core_axis_name="c", subcore_axis_name="s")
    fn = functools.partial(
        pl.kernel,
        out_type=jax.ShapeDtypeStruct((Bn,), _f32),
        mesh=mesh,
        compiler_params=pltpu.CompilerParams(needs_layout_passes=False),
        scratch_types=[
            pltpu.VMEM((1920,), _i32),
            pltpu.VMEM((2 * UPG, 128), _f32),
            pltpu.VMEM((2 * UPG,), _f32),
            pltpu.VMEM((2 * UPG,), _f32),
            pltpu.VMEM((1024,), _f32),
            pltpu.VMEM((UPW, 128), _f32),
            pltpu.VMEM((640,), _i32),
            pltpu.VMEM((640,), _i32),
            pltpu.VMEM((640,), _i32),
            pltpu.VMEM((2048,), _f32),
            pltpu.VMEM((LANES,), _f32),
            pltpu.VMEM((LANES,), _f32),
            pltpu.VMEM((LANES,), _i32),
            pltpu.VMEM((UPW,), _f32),
            pltpu.SemaphoreType.DMA,
        ],
    )(_sc_body)
    return fn(h0, t0, r0, h1, t1, r1, recf, items, wcat, s1t, s3t, s2t, ent4)


# speedup vs baseline: 1.4877x; 1.4877x over previous
import functools

import jax
import jax.numpy as jnp
from jax import lax
from jax.experimental import pallas as pl
from jax.experimental.pallas import tpu as pltpu
from jax.experimental.pallas import tpu_sc as plsc

DIM = 32
NC = 2
NS = 16
NW = NC * NS
LANES = 16
M = 16
LREC = 20
UPW = 32
UPG = LREC * M
CH = 80
QROWS = 250000

_i32 = jnp.int32
_f32 = jnp.float32


def _proj_body(x_ref, wm_ref, *out_refs):
    y = jax.lax.dot_general(wm_ref[...], x_ref[...],
                            (((0,), (1,)), ((), ())),
                            preferred_element_type=_f32)
    for k in range(8):
        out_refs[k][...] = y[k, :]


def _projections(ent4, w1, w3):
    n4 = ent4.shape[0]
    blk = 8192
    grid = (n4 + blk - 1) // blk
    wm = jnp.concatenate(
        [jnp.kron(jnp.eye(4, dtype=_f32), w1[:, None]),
         jnp.kron(jnp.eye(4, dtype=_f32), w3[:, None])], axis=1)
    outs = pl.pallas_call(
        _proj_body,
        grid=(grid,),
        in_specs=[
            pl.BlockSpec((blk, 128), lambda g: (g, 0)),
            pl.BlockSpec((128, 8), lambda g: (0, 0)),
        ],
        out_specs=[pl.BlockSpec((blk,), lambda g: (g,))] * 8,
        out_shape=[jax.ShapeDtypeStruct((n4,), _f32)] * 8,
    )(ent4, wm)
    return jnp.concatenate(outs[:4]), jnp.concatenate(outs[4:])


def _rel_body(x_ref, w2_ref, s2_ref):
    s2_ref[...] = jnp.sum(x_ref[...] * w2_ref[0, :][None, :], axis=1)


def _rel_projection(relp, w2):
    n = relp.shape[0]
    return pl.pallas_call(
        _rel_body,
        out_shape=jax.ShapeDtypeStruct((n,), _f32),
    )(relp, w2.reshape(1, DIM))


def _sc_body(h0, t0, r0, h1, t1, r1, recf, itemsf, wcat, s1t, s3t, s2t, ent4,
             out_hbm, idxb, gtep, s1b, s3b, s2l, itrowp, recb, pidxb, subb,
             hpb, tpb, uemb, wbuf, pbuf, sbuf, outb, sem, semI, semG0, semG1):
    wid = lax.axis_index("s") * NC + lax.axis_index("c")
    iota16 = lax.iota(_i32, LANES)

    pltpu.sync_copy(wcat, wbuf)
    pltpu.sync_copy(s2t, s2l)
    pltpu.sync_copy(recf.at[pl.ds(wid * 640, 640)], recb)
    pltpu.sync_copy(itemsf.at[pl.ds(wid * UPW, UPW)], idxb.at[pl.ds(0, UPW)])

    for g in range(2):
        iv = idxb[pl.ds(g * LANES, LANES)]
        pidxb[pl.ds(g * LANES, LANES)] = lax.shift_right_logical(iv, 2)
        subb[pl.ds(g * LANES, LANES)] = lax.shift_left(
            jnp.bitwise_and(iv, jnp.full((LANES,), 3, _i32)), 5)
    pltpu.async_copy(ent4.at[pidxb.at[pl.ds(0, UPW)]], itrowp, sem).wait()

    def _fitem(u, o):
        subspl = plsc.load_gather(subb, [jnp.full((LANES,), u, _i32)])
        c0 = subspl + iota16
        rowspl = jnp.full((LANES,), u, _i32)
        i0 = plsc.load_gather(itrowp, [rowspl, c0])
        i1 = plsc.load_gather(itrowp, [rowspl, c0 + LANES])
        uemb[pl.ds(1024 + u * DIM, LANES)] = i0
        uemb[pl.ds(1024 + u * DIM + LANES, LANES)] = i1
        return o

    lax.fori_loop(0, UPW, _fitem, 0)

    b_v = wbuf[...]

    def _recsub(g, o):
        iv = recb[pl.ds(g * LANES, LANES)]
        pidxb[pl.ds(g * LANES, LANES)] = lax.shift_right_logical(iv, 2)
        subb[pl.ds(g * LANES, LANES)] = lax.shift_left(
            jnp.bitwise_and(iv, jnp.full((LANES,), 3, _i32)), 5)
        return o

    lax.fori_loop(0, 40, _recsub, 0)
    for half in range(2):
        hs = []
        for c in range(4):
            hs.append(pltpu.async_copy(
                ent4.at[pidxb.at[pl.ds(half * 320 + c * CH, CH)]],
                gtep.at[pl.ds(c * CH, CH)], sem))
        for hnd in hs:
            hnd.wait()

        def _recuser(i, _, half=half):
            def _recl(l, acc):
                row = i * LREC + l
                subspl = plsc.load_gather(
                    subb, [jnp.full((LANES,), half * 320 + row, _i32)])
                c0 = subspl + iota16
                rowspl = jnp.full((LANES,), row, _i32)
                return (acc[0] + plsc.load_gather(gtep, [rowspl, c0]),
                        acc[1] + plsc.load_gather(gtep, [rowspl, c0 + LANES]))

            a0, a1 = lax.fori_loop(
                0, LREC, _recl,
                (jnp.zeros((LANES,), _f32), jnp.zeros((LANES,), _f32)))
            u = half * LANES + i
            uemb[pl.ds(u * DIM, LANES)] = a0
            uemb[pl.ds(u * DIM + LANES, LANES)] = a1
            return _

        lax.fori_loop(0, 16, _recuser, 0)

    heads = (h0, h1)
    tails = (t0, t1)
    rels = (r0, r1)
    c3v = jnp.full((LANES,), 3, _i32)
    cqv = jnp.full((LANES,), QROWS, _i32)

    def _idx_copies(u, ub, fire):
        off = (wid * UPW + u) * UPG
        res = []
        for h in range(2):
            for k, arr in ((0, heads[h]), (1, tails[h]), (2, rels[h])):
                src = arr.at[pl.ds(off, UPG)]
                dst = idxb.at[pl.ds(ub * 1920 + h * 960 + k * 320, UPG)]
                if fire:
                    res.append(pltpu.async_copy(src, dst, semI))
                else:
                    pltpu.make_async_copy(src, dst, semI).wait()
        return res

    def _transform(ub):
        ib = ub * 1920
        pb = ub * 640

        def _pk(g, o):
            h = g // 20
            gg = g - h * 20
            iv = idxb[pl.ds(ib + h * 960 + 320 + gg * LANES, LANES)]
            pk_ = lax.shift_right_logical(iv, 2)
            pidxb[pl.ds(pb + g * LANES, LANES)] = pk_
            subb[pl.ds(pb + g * LANES, LANES)] = lax.shift_left(
                jnp.bitwise_and(iv, c3v), 5)
            tpb[pl.ds(pb + g * LANES, LANES)] = (
                jnp.bitwise_and(iv, c3v) * cqv + pk_)
            hv = idxb[pl.ds(ib + h * 960 + gg * LANES, LANES)]
            hpb[pl.ds(pb + g * LANES, LANES)] = (
                jnp.bitwise_and(hv, c3v) * cqv + lax.shift_right_logical(hv, 2))
            return o

        lax.fori_loop(0, 40, _pk, 0)

    def _gathers(ub, h, sg, fire):
        res = []
        pb = ub * 640
        for c in range(4):
            trip = [
                (ent4.at[pidxb.at[pl.ds(pb + h * 320 + c * CH, CH)]],
                 gtep.at[pl.ds(h * 320 + c * CH, CH)]),
                (s1t.at[hpb.at[pl.ds(pb + h * 320 + c * CH, CH)]],
                 s1b.at[pl.ds(h * 320 + c * CH, CH)]),
                (s3t.at[tpb.at[pl.ds(pb + h * 320 + c * CH, CH)]],
                 s3b.at[pl.ds(h * 320 + c * CH, CH)]),
            ]
            for s_, d_ in trip:
                if fire:
                    res.append(pltpu.async_copy(s_, d_, sg))
                else:
                    pltpu.make_async_copy(s_, d_, sg).wait()
        return res

    def _hop(ub, h, acc):
        def _lstep(l, carry):
            a0, a1 = carry
            rbase = l * M
            s1 = s1b[pl.ds(h * 320 + rbase, M)]
            s3 = s3b[pl.ds(h * 320 + rbase, M)]
            relidx = idxb[pl.ds(ub * 1920 + h * 960 + 640 + rbase, M)]
            s2 = plsc.load_gather(s2l, [relidx])
            logit = s1 + s2 + s3 + b_v
            sig = 1.0 / (1.0 + jnp.exp(-logit))
            e = jnp.exp(sig)
            p = e / jnp.sum(e)
            pbuf[...] = p
            sbuf[...] = subb[pl.ds(ub * 640 + h * 320 + rbase, M)]

            def _mstep(mj, c3):
                b0, b1 = c3
                for q in range(2):
                    m = mj * 2 + q
                    mspl = jnp.full((LANES,), m, _i32)
                    pm = plsc.load_gather(pbuf, [mspl])
                    c0 = plsc.load_gather(sbuf, [mspl]) + iota16
                    rowspl = jnp.full((LANES,), h * 320 + rbase + m, _i32)
                    b0 = b0 + pm * plsc.load_gather(gtep, [rowspl, c0])
                    b1 = b1 + pm * plsc.load_gather(gtep, [rowspl, c0 + LANES])
                return (b0, b1)

            return lax.fori_loop(0, 8, _mstep, (a0, a1))

        return lax.fori_loop(0, LREC, _lstep, acc)

    for hnd in _idx_copies(0, 0, True):
        hnd.wait()
    _transform(0)
    _gathers(0, 0, semG0, True)

    def _user(u_loc, _):
        up = jnp.bitwise_and(u_loc, 1)
        upn = 1 - up
        _gathers(up, 1, semG1, True)

        @pl.when(u_loc < UPW - 1)
        def _fire_idx():
            _idx_copies(u_loc + 1, upn, True)

        _gathers(up, 0, semG0, False)
        acc = (uemb[pl.ds(u_loc * DIM, LANES)],
               uemb[pl.ds(u_loc * DIM + LANES, LANES)])
        acc = _hop(up, 0, acc)

        @pl.when(u_loc < UPW - 1)
        def _stage_next():
            _idx_copies(u_loc + 1, upn, False)
            _transform(upn)
            _gathers(upn, 0, semG0, True)

        _gathers(up, 1, semG1, False)
        acc = _hop(up, 1, acc)
        uemb[pl.ds(u_loc * DIM, LANES)] = acc[0]
        uemb[pl.ds(u_loc * DIM + LANES, LANES)] = acc[1]
        return _

    lax.fori_loop(0, UPW, _user, 0)

    for g in range(2):

        def _fu(i, o, g=g):
            u = g * LANES + i
            a0 = uemb[pl.ds(u * DIM, LANES)]
            a1 = uemb[pl.ds(u * DIM + LANES, LANES)]
            i0 = uemb[pl.ds(1024 + u * DIM, LANES)]
            i1 = uemb[pl.ds(1024 + u * DIM + LANES, LANES)]
            dot = jnp.sum(a0 * i0 + a1 * i1)
            return jnp.where(iota16 == i, dot, o)

        dots = lax.fori_loop(0, LANES, _fu, jnp.zeros((LANES,), _f32))
        outb[pl.ds(g * LANES, LANES)] = 1.0 / (1.0 + jnp.exp(-dots))

    pltpu.sync_copy(outb, out_hbm.at[pl.ds(wid * UPW, UPW)])


def kernel(pairs, records, ripple_heads, ripple_relations, ripple_tails,
           entity_emb, relation_emb, W_w, W_b):
    Bn, Ln = records.shape
    items = pairs[:, 1].astype(_i32)
    recf = records.reshape(-1).astype(_i32)
    h0 = ripple_heads[0].reshape(-1).astype(_i32)
    h1 = ripple_heads[1].reshape(-1).astype(_i32)
    t0 = ripple_tails[0].reshape(-1).astype(_i32)
    t1 = ripple_tails[1].reshape(-1).astype(_i32)
    r0 = ripple_relations[0].reshape(-1).astype(_i32)
    r1 = ripple_relations[1].reshape(-1).astype(_i32)
    wcat = jnp.full((LANES,), W_b[0], _f32)
    relp = jnp.pad(relation_emb, ((0, 1024 - relation_emb.shape[0]), (0, 0)))

    ent4 = entity_emb.reshape(250000, 128)
    s1t, s3t = _projections(ent4, W_w[:DIM, 0], W_w[2 * DIM:, 0])
    s2t = _rel_projection(relp, W_w[DIM:2 * DIM, 0])

    mesh = plsc.VectorSubcoreMesh(core_axis_name="c", subcore_axis_name="s")
    fn = functools.partial(
        pl.kernel,
        out_type=jax.ShapeDtypeStruct((Bn,), _f32),
        mesh=mesh,
        compiler_params=pltpu.CompilerParams(needs_layout_passes=False),
        scratch_types=[
            pltpu.VMEM((3840,), _i32),
            pltpu.VMEM((2 * UPG, 128), _f32),
            pltpu.VMEM((2 * UPG,), _f32),
            pltpu.VMEM((2 * UPG,), _f32),
            pltpu.VMEM((1024,), _f32),
            pltpu.VMEM((UPW, 128), _f32),
            pltpu.VMEM((640,), _i32),
            pltpu.VMEM((1280,), _i32),
            pltpu.VMEM((1280,), _i32),
            pltpu.VMEM((1280,), _i32),
            pltpu.VMEM((1280,), _i32),
            pltpu.VMEM((2048,), _f32),
            pltpu.VMEM((LANES,), _f32),
            pltpu.VMEM((LANES,), _f32),
            pltpu.VMEM((LANES,), _i32),
            pltpu.VMEM((UPW,), _f32),
            pltpu.SemaphoreType.DMA,
            pltpu.SemaphoreType.DMA,
            pltpu.SemaphoreType.DMA,
            pltpu.SemaphoreType.DMA,
        ],
    )(_sc_body)
    return fn(h0, t0, r0, h1, t1, r1, recf, items, wcat, s1t, s3t, s2t, ent4)
